# baseline (speedup 1.0000x reference)
import jax
import jax.numpy as jnp
from jax.experimental import pallas as pl
from jax.experimental.pallas import tpu as pltpu

N = 4096
FIN = 128
NBOND = 4
FOUT = 128
EPS = 1e-5
BM = 256


def _body(x_ref, g_ref, be_ref, w_ref, bias_ref, bi_ref, out_ref, h_ref):
    i = pl.program_id(0)
    k = pl.program_id(1)

    @pl.when((i == 0) & (k == 0))
    def _compute_h():
        x = x_ref[...]
        mean = jnp.mean(x, axis=0, keepdims=True)
        var = jnp.mean((x - mean) ** 2, axis=0, keepdims=True)
        hn = (x - mean) / jnp.sqrt(var + EPS) * g_ref[...] + be_ref[...]
        ha = jnp.where(hn > 0, hn, jnp.exp(jnp.minimum(hn, 0.0)) - 1.0)
        h_ref[...] = jax.lax.dot_general(
            ha, w_ref[...], (((1,), (1,)), ((), ())),
            preferred_element_type=jnp.float32,
        ) + bias_ref[...]

    contrib = jax.lax.dot_general(
        bi_ref[...], h_ref[:, pl.ds(k * FOUT, FOUT)],
        (((1,), (0,)), ((), ())),
        preferred_element_type=jnp.float32,
    )

    @pl.when(k == 0)
    def _init():
        out_ref[...] = contrib

    @pl.when(k > 0)
    def _acc():
        out_ref[...] += contrib


def kernel(atom_features, bond_info, bn_gamma, bn_beta, W, b):
    grid = (N // BM, NBOND)
    return pl.pallas_call(
        _body,
        grid=grid,
        in_specs=[
            pl.BlockSpec((N, FIN), lambda i, k: (0, 0)),
            pl.BlockSpec((1, FIN), lambda i, k: (0, 0)),
            pl.BlockSpec((1, FIN), lambda i, k: (0, 0)),
            pl.BlockSpec((NBOND * FOUT, FIN), lambda i, k: (0, 0)),
            pl.BlockSpec((1, NBOND * FOUT), lambda i, k: (0, 0)),
            pl.BlockSpec((BM, N), lambda i, k: (i, k)),
        ],
        out_specs=pl.BlockSpec((BM, FOUT), lambda i, k: (i, 0)),
        out_shape=jax.ShapeDtypeStruct((N, FOUT), jnp.float32),
        scratch_shapes=[pltpu.VMEM((N, NBOND * FOUT), jnp.float32)],
    )(
        atom_features,
        bn_gamma.reshape(1, FIN),
        bn_beta.reshape(1, FIN),
        W,
        b.reshape(1, NBOND * FOUT),
        bond_info,
    )


# speedup vs baseline: 1.2113x; 1.2113x over previous
import jax
import jax.numpy as jnp
from jax.experimental import pallas as pl
from jax.experimental.pallas import tpu as pltpu

N = 4096
FIN = 128
NBOND = 4
FOUT = 128
EPS = 1e-5
BM = 1024


def _body(x_ref, g_ref, be_ref, w_ref, bias_ref, bi_ref, out_ref, h_ref):
    i = pl.program_id(0)
    k = pl.program_id(1)

    @pl.when((i == 0) & (k == 0))
    def _compute_h():
        x = x_ref[...]
        mean = jnp.mean(x, axis=0, keepdims=True)
        var = jnp.mean((x - mean) ** 2, axis=0, keepdims=True)
        hn = (x - mean) / jnp.sqrt(var + EPS) * g_ref[...] + be_ref[...]
        ha = jnp.where(hn > 0, hn, jnp.exp(jnp.minimum(hn, 0.0)) - 1.0)
        h_ref[...] = jax.lax.dot_general(
            ha, w_ref[...], (((1,), (1,)), ((), ())),
            preferred_element_type=jnp.float32,
        ) + bias_ref[...]

    contrib = jax.lax.dot_general(
        bi_ref[...], h_ref[:, pl.ds(k * FOUT, FOUT)],
        (((1,), (0,)), ((), ())),
        preferred_element_type=jnp.float32,
    )

    @pl.when(k == 0)
    def _init():
        out_ref[...] = contrib

    @pl.when(k > 0)
    def _acc():
        out_ref[...] += contrib


def kernel(atom_features, bond_info, bn_gamma, bn_beta, W, b):
    grid = (N // BM, NBOND)
    return pl.pallas_call(
        _body,
        grid=grid,
        in_specs=[
            pl.BlockSpec((N, FIN), lambda i, k: (0, 0)),
            pl.BlockSpec((1, FIN), lambda i, k: (0, 0)),
            pl.BlockSpec((1, FIN), lambda i, k: (0, 0)),
            pl.BlockSpec((NBOND * FOUT, FIN), lambda i, k: (0, 0)),
            pl.BlockSpec((1, NBOND * FOUT), lambda i, k: (0, 0)),
            pl.BlockSpec((BM, N), lambda i, k: (i, k)),
        ],
        out_specs=pl.BlockSpec((BM, FOUT), lambda i, k: (i, 0)),
        out_shape=jax.ShapeDtypeStruct((N, FOUT), jnp.float32),
        scratch_shapes=[pltpu.VMEM((N, NBOND * FOUT), jnp.float32)],
    )(
        atom_features,
        bn_gamma.reshape(1, FIN),
        bn_beta.reshape(1, FIN),
        W,
        b.reshape(1, NBOND * FOUT),
        bond_info,
    )


# speedup vs baseline: 1.2287x; 1.0144x over previous
import jax
import jax.numpy as jnp
from jax.experimental import pallas as pl
from jax.experimental.pallas import tpu as pltpu

N = 4096
FIN = 128
NBOND = 4
FOUT = 128
EPS = 1e-5
BM = 512


def _body(x_ref, g_ref, be_ref, w_ref, bias_ref, bi_ref, out_ref, h_ref):
    i = pl.program_id(0)
    k = pl.program_id(1)

    @pl.when((i == 0) & (k == 0))
    def _compute_h():
        x = x_ref[...]
        mean = jnp.mean(x, axis=0, keepdims=True)
        var = jnp.mean((x - mean) ** 2, axis=0, keepdims=True)
        hn = (x - mean) / jnp.sqrt(var + EPS) * g_ref[...] + be_ref[...]
        ha = jnp.where(hn > 0, hn, jnp.exp(jnp.minimum(hn, 0.0)) - 1.0)
        h_ref[...] = jax.lax.dot_general(
            ha, w_ref[...], (((1,), (1,)), ((), ())),
            preferred_element_type=jnp.float32,
        ) + bias_ref[...]

    for b in range(NBOND):
        @pl.when(k == b)
        def _step(b=b):
            contrib = jax.lax.dot_general(
                bi_ref[...], h_ref[:, b * FOUT:(b + 1) * FOUT],
                (((1,), (0,)), ((), ())),
                preferred_element_type=jnp.float32,
            )
            if b == 0:
                out_ref[...] = contrib
            else:
                out_ref[...] += contrib


def kernel(atom_features, bond_info, bn_gamma, bn_beta, W, b):
    grid = (N // BM, NBOND)
    return pl.pallas_call(
        _body,
        grid=grid,
        in_specs=[
            pl.BlockSpec((N, FIN), lambda i, k: (0, 0)),
            pl.BlockSpec((1, FIN), lambda i, k: (0, 0)),
            pl.BlockSpec((1, FIN), lambda i, k: (0, 0)),
            pl.BlockSpec((NBOND * FOUT, FIN), lambda i, k: (0, 0)),
            pl.BlockSpec((1, NBOND * FOUT), lambda i, k: (0, 0)),
            pl.BlockSpec((BM, N), lambda i, k: (i, k)),
        ],
        out_specs=pl.BlockSpec((BM, FOUT), lambda i, k: (i, 0)),
        out_shape=jax.ShapeDtypeStruct((N, FOUT), jnp.float32),
        scratch_shapes=[pltpu.VMEM((N, NBOND * FOUT), jnp.float32)],
    )(
        atom_features,
        bn_gamma.reshape(1, FIN),
        bn_beta.reshape(1, FIN),
        W,
        b.reshape(1, NBOND * FOUT),
        bond_info,
    )
